# baseline (speedup 1.0000x reference)
import jax
import jax.numpy as jnp
from jax.experimental import pallas as pl
from jax.experimental.pallas import tpu as pltpu


def _body(a_ref, b_ref, m_ref, x_ref, o_ref):
    a = a_ref[0]
    b = b_ref[0]
    m = m_ref[...]
    scale = 1.0 + m * (a - 1.0)
    bias = m * b
    o_ref[...] = x_ref[...] * scale + bias


def kernel(inputs, mask, alpha, beta):
    B, C = inputs.shape
    xt = inputs.T
    mf2 = jnp.where(mask[:, None], 1.0, 0.0)
    blk = 2048
    out_t = pl.pallas_call(
        _body,
        grid=(B // blk,),
        in_specs=[
            pl.BlockSpec(memory_space=pltpu.SMEM),
            pl.BlockSpec(memory_space=pltpu.SMEM),
            pl.BlockSpec((C, 1), lambda i: (0, 0)),
            pl.BlockSpec((C, blk), lambda i: (0, i)),
        ],
        out_specs=pl.BlockSpec((C, blk), lambda i: (0, i)),
        out_shape=jax.ShapeDtypeStruct((C, B), jnp.float32),
    )(alpha, beta, mf2, xt)
    return out_t.T
